# baseline (speedup 1.0000x reference)
import functools
import jax
import jax.numpy as jnp
from jax import lax
from jax.experimental import pallas as pl
from jax.experimental.pallas import tpu as pltpu
from jax.experimental.pallas import tpu_sc as plsc


def _bcast_gather(v, idx):
    dn = lax.GatherDimensionNumbers(
        offset_dims=(), collapsed_slice_dims=(0,), start_index_map=(0,))
    return lax.gather(v, idx.reshape(idx.shape[0], 1), dn, slice_sizes=(1,),
                      mode=lax.GatherScatterMode.PROMISE_IN_BOUNDS)

_K = 256
_N = 4096
_BIG = 1e30
_NS = 16
_RB = _K // _NS
_B = 128
_NBLK = _N // _B
_STEPS = _NBLK + _NS - 1


def _sc_body(x_hbm, k_hbm, out_hbm, xv, kv, nbv, botv, shared):
    sid = lax.axis_index("s")
    iota = lax.iota(jnp.int32, _NS)
    shift_idx = jnp.maximum(iota - 1, 0)
    fifteens = jnp.full((_NS,), _NS - 1, jnp.int32)
    mask0 = iota == 0
    zeros = jnp.zeros((_NS,), jnp.float32)
    ones = zeros + 1.0
    selv = jnp.where(sid == 0, zeros, ones)
    top0 = jnp.where(sid == 0, zeros, zeros + _BIG)

    pltpu.sync_copy(k_hbm.at[pl.ds(sid * _RB, _RB)], kv)
    kvreg = kv[...]

    def step(s, carry):
        dprev, topv = carry
        b = jnp.clip(s - sid, 0, _NBLK - 1)
        jb = b * _B
        pltpu.sync_copy(x_hbm.at[pl.ds(jb, _B)], xv)
        pltpu.sync_copy(shared.at[(sid - 1) % _NS], nbv)
        plsc.subcore_barrier()

        isfirst = s == sid
        dprev = jnp.where(isfirst, zeros + _BIG, dprev)
        topv = jnp.where(isfirst, top0, topv)

        for g in range(_B // _NS):
            xg = xv[pl.ds(g * _NS, _NS)]
            ng = nbv[pl.ds(g * _NS, _NS)]

            def col(jr, c2, xg=xg, ng=ng):
                dprev, topv, bot_acc = c2
                jrv = jnp.full((_NS,), jr, jnp.int32)
                shp = _bcast_gather(dprev, shift_idx)
                shp = jnp.where(mask0, topv, shp)
                xjv = _bcast_gather(xg, jrv)
                dm1 = _bcast_gather(ng, jrv) * selv
                c = kvreg - xjv
                c = c * c
                sv = c + jnp.minimum(dprev, shp)
                cc = plsc.cumsum(c)
                e = -plsc.cummax(cc - sv)
                e = jnp.minimum(e, dm1)
                d = cc + e
                d15 = _bcast_gather(d, fifteens)
                bot_acc = jnp.where(iota == jr, d15, bot_acc)
                return d, dm1, bot_acc

            dprev, topv, bot_acc = lax.fori_loop(
                0, _NS, col, (dprev, topv, zeros))
            botv[pl.ds(g * _NS, _NS)] = bot_acc

        pltpu.sync_copy(botv, shared.at[sid])

        @pl.when(sid == _NS - 1)
        def _():
            pltpu.sync_copy(botv, out_hbm.at[pl.ds(jb, _B)])

        plsc.subcore_barrier()
        return dprev, topv

    lax.fori_loop(0, _STEPS, step, (zeros + _BIG, top0))


def _run(x, kern):
    f = functools.partial(
        pl.kernel,
        out_type=jax.ShapeDtypeStruct((_N,), jnp.float32),
        mesh=plsc.VectorSubcoreMesh(core_axis_name="c", subcore_axis_name="s"),
        compiler_params=pltpu.CompilerParams(needs_layout_passes=False),
        scratch_types=[
            pltpu.VMEM((_B,), jnp.float32),
            pltpu.VMEM((_RB,), jnp.float32),
            pltpu.VMEM((_B,), jnp.float32),
            pltpu.VMEM((_B,), jnp.float32),
            pltpu.VMEM_SHARED((_NS, _B), jnp.float32),
        ],
    )(_sc_body)
    return f(x, kern)


def kernel(x, kernel):
    return _run(x, kernel)
